# baseline (speedup 1.0000x reference)
import jax
import jax.numpy as jnp
from jax import lax
from jax.experimental import pallas as pl
from jax.experimental.pallas import tpu as pltpu
from jax.experimental.pallas import tpu_sc as plsc

_V = 1_000_000
_VT = 7813
_VP = _VT * 128
_D = 32
_NI = 16384
_NJ = 26

_NC = 2
_NS = 16
_NW = _NC * _NS

_T_PER_W = 245
_OUTER = 123

_I_PER_W = _NI // _NW
_IC = 4
_CH = _IC * _NJ
_CHUNKS = _I_PER_W // _IC


def _transpose_chunk(x_v, y_v):
    iota = lax.broadcasted_iota(jnp.int32, (16,), 0)
    qo = lax.shift_right_logical(iota, 2)
    pending = None
    for d0 in (0, 16):
        for k in range(16):
            rows = d0 + jnp.bitwise_and(iota + k, 15)
            lane = 32 * jnp.bitwise_and(iota, 3) + rows
            vs = [plsc.load_gather(x_v, [rows, iota + 16 * s])
                  for s in range(8)]
            if pending is not None:
                p_lane, p_vs = pending
                for s in range(8):
                    plsc.store_scatter(y_v, [qo + 4 * s, p_lane], p_vs[s])
            pending = (lane, vs)
    p_lane, p_vs = pending
    for s in range(8):
        plsc.store_scatter(y_v, [qo + 4 * s, p_lane], p_vs[s])


def _repack_body(pt_hbm, tbl_hbm, x_va, x_vb, y_va, y_vb,
                 isem_a, isem_b, osem_a, osem_b):
    wid = lax.axis_index("s") * _NC + lax.axis_index("c")
    c_base = wid * _T_PER_W

    def in_copy(c, x_v, isem):
        col = jnp.minimum(c, _VT - 1) * 128
        return pltpu.async_copy(pt_hbm.at[:, pl.ds(col, 128)], x_v, isem)

    def out_copy(c, y_v, osem):
        row = jnp.minimum(c, _VT - 1) * 32
        return pltpu.async_copy(y_v, tbl_hbm.at[pl.ds(row, 32)], osem)

    in_copy(c_base, x_va, isem_a)
    in_copy(c_base + 1, x_vb, isem_b)

    def body(m, carry):
        ca = c_base + 2 * m
        cb = ca + 1
        pltpu.make_async_copy(pt_hbm.at[:, pl.ds(0, 128)], x_va, isem_a).wait()

        @pl.when(m > 0)
        def _():
            pltpu.make_async_copy(
                y_va, tbl_hbm.at[pl.ds(0, 32)], osem_a).wait()

        _transpose_chunk(x_va, y_va)
        in_copy(ca + 2, x_va, isem_a)
        out_copy(ca, y_va, osem_a)
        pltpu.make_async_copy(pt_hbm.at[:, pl.ds(0, 128)], x_vb, isem_b).wait()

        @pl.when(m > 0)
        def _():
            pltpu.make_async_copy(
                y_vb, tbl_hbm.at[pl.ds(0, 32)], osem_b).wait()

        _transpose_chunk(x_vb, y_vb)
        in_copy(cb + 2, x_vb, isem_b)
        out_copy(cb, y_vb, osem_b)
        return carry

    lax.fori_loop(0, _OUTER, body, 0)
    pltpu.make_async_copy(pt_hbm.at[:, pl.ds(0, 128)], x_va, isem_a).wait()
    pltpu.make_async_copy(pt_hbm.at[:, pl.ds(0, 128)], x_vb, isem_b).wait()
    pltpu.make_async_copy(y_va, tbl_hbm.at[pl.ds(0, 32)], osem_a).wait()
    pltpu.make_async_copy(y_vb, tbl_hbm.at[pl.ds(0, 32)], osem_b).wait()


def _gather_body(table_hbm, idx_hbm, out_hbm, idx_v, rows_a, rows_b,
                 gsem_a, gsem_b, wsem_a, wsem_b):
    wid = lax.axis_index("s") * _NC + lax.axis_index("c")
    pltpu.sync_copy(idx_hbm.at[pl.ds(wid * _CHUNKS, _CHUNKS)], idx_v)
    i_base = wid * _I_PER_W

    def write_out(rows_v, c, wsem):
        i0 = i_base + c * _IC
        return [
            pltpu.async_copy(
                rows_v.at[pl.ds(k * _NJ, _NJ)],
                out_hbm.at[i0 + k, pl.ds(0, _NJ), pl.ds(0, _D)],
                wsem)
            for k in range(_IC)
        ]

    def body(m, carry):
        c0 = 2 * m
        ga = pltpu.async_copy(table_hbm.at[idx_v.at[c0]], rows_a, gsem_a)
        gb = pltpu.async_copy(table_hbm.at[idx_v.at[c0 + 1]], rows_b, gsem_b)
        ga.wait()
        wa = write_out(rows_a, c0, wsem_a)
        gb.wait()
        wb = write_out(rows_b, c0 + 1, wsem_b)
        for d in wa:
            d.wait()
        for d in wb:
            d.wait()
        return carry

    lax.fori_loop(0, _CHUNKS // 2, body, 0)


@jax.jit
def _repack(pt):
    mesh = plsc.VectorSubcoreMesh(core_axis_name="c", subcore_axis_name="s")
    return pl.kernel(
        _repack_body,
        out_type=jax.ShapeDtypeStruct((_VP // 4, 128), jnp.float32),
        mesh=mesh,
        scratch_types=[
            pltpu.VMEM((_D, 128), jnp.float32),
            pltpu.VMEM((_D, 128), jnp.float32),
            pltpu.VMEM((32, 128), jnp.float32),
            pltpu.VMEM((32, 128), jnp.float32),
            pltpu.SemaphoreType.DMA,
            pltpu.SemaphoreType.DMA,
            pltpu.SemaphoreType.DMA,
            pltpu.SemaphoreType.DMA,
        ],
        compiler_params=pltpu.CompilerParams(
            use_tc_tiling_on_sc=True,
            needs_layout_passes=False,
            disable_bounds_checks=True,
        ),
    )(pt)


@jax.jit
def _gather(tbl, idx2d):
    mesh = plsc.VectorSubcoreMesh(core_axis_name="c", subcore_axis_name="s")
    return pl.kernel(
        _gather_body,
        out_type=jax.ShapeDtypeStruct((_NI, 32, 128), jnp.float32),
        mesh=mesh,
        scratch_types=[
            pltpu.VMEM((_CHUNKS, _CH), jnp.int32),
            pltpu.VMEM((_CH, _D), jnp.float32),
            pltpu.VMEM((_CH, _D), jnp.float32),
            pltpu.SemaphoreType.DMA,
            pltpu.SemaphoreType.DMA,
            pltpu.SemaphoreType.DMA,
            pltpu.SemaphoreType.DMA,
        ],
        compiler_params=pltpu.CompilerParams(use_tc_tiling_on_sc=False),
    )(tbl, idx2d)


def kernel(params, indices):
    pt = params.T
    tbl = _repack(pt).reshape(_VP, _D)
    idx2d = indices.reshape(-1).astype(jnp.int32).reshape(_NW * _CHUNKS, _CH)
    out_big = _gather(tbl, idx2d)
    return out_big[:, :_NJ, :_D]


# speedup vs baseline: 1.0066x; 1.0066x over previous
import jax
import jax.numpy as jnp
from jax import lax
from jax.experimental import pallas as pl
from jax.experimental.pallas import tpu as pltpu
from jax.experimental.pallas import tpu_sc as plsc

_V = 1_000_000
_VT = 7813
_VP = _VT * 128
_D = 32
_NI = 16384
_NJ = 26

_NC = 2
_NS = 16
_NW = _NC * _NS

_T_PER_W = 245
_OUTER = 123

_I_PER_W = _NI // _NW
_IC = 4
_CH = _IC * _NJ
_CHUNKS = _I_PER_W // _IC


def _transpose_chunk(x_v, y_v):
    iota = lax.broadcasted_iota(jnp.int32, (16,), 0)
    qo = lax.shift_right_logical(iota, 2)
    for d0 in (0, 16):
        for k in range(16):
            rows = d0 + jnp.bitwise_and(iota + k, 15)
            lane = 32 * jnp.bitwise_and(iota, 3) + rows
            vs = [plsc.load_gather(x_v, [rows, iota + 16 * s])
                  for s in range(8)]
            for s in range(8):
                plsc.store_scatter(y_v, [qo + 4 * s, lane], vs[s])


def _repack_body(pt_hbm, tbl_hbm, x_va, x_vb, y_va, y_vb,
                 isem_a, isem_b, osem_a, osem_b):
    wid = lax.axis_index("s") * _NC + lax.axis_index("c")
    c_base = wid * _T_PER_W

    def in_copy(c, x_v, isem):
        col = jnp.minimum(c, _VT - 1) * 128
        return pltpu.async_copy(pt_hbm.at[:, pl.ds(col, 128)], x_v, isem)

    def out_copy(c, y_v, osem):
        row = jnp.minimum(c, _VT - 1) * 32
        return pltpu.async_copy(y_v, tbl_hbm.at[pl.ds(row, 32)], osem)

    in_copy(c_base, x_va, isem_a)
    in_copy(c_base + 1, x_vb, isem_b)

    def body(m, carry):
        ca = c_base + 2 * m
        cb = ca + 1
        pltpu.make_async_copy(pt_hbm.at[:, pl.ds(0, 128)], x_va, isem_a).wait()

        @pl.when(m > 0)
        def _():
            pltpu.make_async_copy(
                y_va, tbl_hbm.at[pl.ds(0, 32)], osem_a).wait()

        _transpose_chunk(x_va, y_va)
        in_copy(ca + 2, x_va, isem_a)
        out_copy(ca, y_va, osem_a)
        pltpu.make_async_copy(pt_hbm.at[:, pl.ds(0, 128)], x_vb, isem_b).wait()

        @pl.when(m > 0)
        def _():
            pltpu.make_async_copy(
                y_vb, tbl_hbm.at[pl.ds(0, 32)], osem_b).wait()

        _transpose_chunk(x_vb, y_vb)
        in_copy(cb + 2, x_vb, isem_b)
        out_copy(cb, y_vb, osem_b)
        return carry

    lax.fori_loop(0, _OUTER, body, 0)
    pltpu.make_async_copy(pt_hbm.at[:, pl.ds(0, 128)], x_va, isem_a).wait()
    pltpu.make_async_copy(pt_hbm.at[:, pl.ds(0, 128)], x_vb, isem_b).wait()
    pltpu.make_async_copy(y_va, tbl_hbm.at[pl.ds(0, 32)], osem_a).wait()
    pltpu.make_async_copy(y_vb, tbl_hbm.at[pl.ds(0, 32)], osem_b).wait()


def _gather_body(table_hbm, idx_hbm, out_hbm, idx_v, rows_a, rows_b,
                 gsem_a, gsem_b, wsem_a, wsem_b):
    wid = lax.axis_index("s") * _NC + lax.axis_index("c")
    pltpu.sync_copy(idx_hbm.at[pl.ds(wid * _CHUNKS, _CHUNKS)], idx_v)
    i_base = wid * _I_PER_W

    def write_out(rows_v, c, wsem):
        i0 = i_base + c * _IC
        return [
            pltpu.async_copy(
                rows_v.at[pl.ds(k * _NJ, _NJ)],
                out_hbm.at[i0 + k, pl.ds(0, _NJ), pl.ds(0, _D)],
                wsem)
            for k in range(_IC)
        ]

    def body(m, carry):
        c0 = 2 * m
        ga = pltpu.async_copy(table_hbm.at[idx_v.at[c0]], rows_a, gsem_a)
        gb = pltpu.async_copy(table_hbm.at[idx_v.at[c0 + 1]], rows_b, gsem_b)
        ga.wait()
        wa = write_out(rows_a, c0, wsem_a)
        gb.wait()
        wb = write_out(rows_b, c0 + 1, wsem_b)
        for d in wa:
            d.wait()
        for d in wb:
            d.wait()
        return carry

    lax.fori_loop(0, _CHUNKS // 2, body, 0)


@jax.jit
def _repack(pt):
    mesh = plsc.VectorSubcoreMesh(core_axis_name="c", subcore_axis_name="s")
    return pl.kernel(
        _repack_body,
        out_type=jax.ShapeDtypeStruct((_VP // 4, 128), jnp.float32),
        mesh=mesh,
        scratch_types=[
            pltpu.VMEM((_D, 128), jnp.float32),
            pltpu.VMEM((_D, 128), jnp.float32),
            pltpu.VMEM((32, 128), jnp.float32),
            pltpu.VMEM((32, 128), jnp.float32),
            pltpu.SemaphoreType.DMA,
            pltpu.SemaphoreType.DMA,
            pltpu.SemaphoreType.DMA,
            pltpu.SemaphoreType.DMA,
        ],
        compiler_params=pltpu.CompilerParams(
            use_tc_tiling_on_sc=True,
            needs_layout_passes=False,
            disable_bounds_checks=True,
        ),
    )(pt)


@jax.jit
def _gather(tbl, idx2d):
    mesh = plsc.VectorSubcoreMesh(core_axis_name="c", subcore_axis_name="s")
    return pl.kernel(
        _gather_body,
        out_type=jax.ShapeDtypeStruct((_NI, 32, 128), jnp.float32),
        mesh=mesh,
        scratch_types=[
            pltpu.VMEM((_CHUNKS, _CH), jnp.int32),
            pltpu.VMEM((_CH, _D), jnp.float32),
            pltpu.VMEM((_CH, _D), jnp.float32),
            pltpu.SemaphoreType.DMA,
            pltpu.SemaphoreType.DMA,
            pltpu.SemaphoreType.DMA,
            pltpu.SemaphoreType.DMA,
        ],
        compiler_params=pltpu.CompilerParams(use_tc_tiling_on_sc=False),
    )(tbl, idx2d)


def kernel(params, indices):
    pt = params.T
    tbl = _repack(pt).reshape(_VP, _D)
    idx2d = indices.reshape(-1).astype(jnp.int32).reshape(_NW * _CHUNKS, _CH)
    out_big = _gather(tbl, idx2d)
    return out_big[:, :_NJ, :_D]
